# Initial kernel scaffold; baseline (speedup 1.0000x reference)
import functools

import jax
import jax.numpy as jnp
from jax import lax
from jax.experimental import pallas as pl
from jax.experimental.pallas import tpu as pltpu
from jax.experimental.pallas import tpu_sc as plsc

BATCH = 4096
HIST = 200
D = 64
LANES = 16

_NC = 2
_NS = 16
_NW = _NC * _NS
_BPW = BATCH // _NW
_G0 = 128
_G1 = HIST - _G0
_UNROLL = 8


def _emb_body(items_hbm, table_hbm, out_hbm, idx_v, rows0, rows1, out_v,
              sem0, sem1):
    wid = lax.axis_index("s") * _NC + lax.axis_index("c")
    base = wid * (_BPW * HIST)

    pltpu.sync_copy(items_hbm.at[pl.ds(base, _BPW * HIST)], idx_v)

    def gather_descs(b, rows, sem):
        srcA = table_hbm.at[idx_v.at[pl.ds(b * HIST, _G0)]]
        srcB = table_hbm.at[idx_v.at[pl.ds(b * HIST + _G0, _G1)]]
        return ((srcA, rows.at[pl.ds(0, _G0)], sem),
                (srcB, rows.at[pl.ds(_G0, _G1)], sem))

    def start_gather(b, rows, sem):
        for desc in gather_descs(b, rows, sem):
            pltpu.async_copy(*desc)

    def wait_gather(b, rows, sem):
        for desc in gather_descs(b, rows, sem):
            pltpu.make_async_copy(*desc).wait()

    def phase(b, rows, sem):
        wait_gather(b, rows, sem)

        def acc_step(i, acc):
            a0, a1, a2, a3 = acc
            for j in range(_UNROLL):
                l = i * _UNROLL + j
                a0 = a0 + rows[l, pl.ds(0, LANES)]
                a1 = a1 + rows[l, pl.ds(LANES, LANES)]
                a2 = a2 + rows[l, pl.ds(2 * LANES, LANES)]
                a3 = a3 + rows[l, pl.ds(3 * LANES, LANES)]
            return (a0, a1, a2, a3)

        zero = jnp.zeros((LANES,), jnp.float32)
        acc = lax.fori_loop(0, HIST // _UNROLL, acc_step,
                            (zero, zero, zero, zero))
        scale = jnp.float32(1.0 / HIST)
        out_v[b, pl.ds(0, LANES)] = acc[0] * scale
        out_v[b, pl.ds(LANES, LANES)] = acc[1] * scale
        out_v[b, pl.ds(2 * LANES, LANES)] = acc[2] * scale
        out_v[b, pl.ds(3 * LANES, LANES)] = acc[3] * scale

        @pl.when(b + 2 < _BPW)
        def _():
            start_gather(b + 2, rows, sem)

    start_gather(0, rows0, sem0)
    start_gather(1, rows1, sem1)

    @pl.loop(0, _BPW // 2)
    def _(g):
        phase(2 * g, rows0, sem0)
        phase(2 * g + 1, rows1, sem1)

    pltpu.sync_copy(out_v, out_hbm.at[pl.ds(wid * _BPW, _BPW)])


@functools.partial(
    pl.kernel,
    out_type=jax.ShapeDtypeStruct((BATCH, D), jnp.float32),
    mesh=plsc.VectorSubcoreMesh(core_axis_name="c", subcore_axis_name="s"),
    scratch_types=[
        pltpu.VMEM((_BPW * HIST,), jnp.int32),
        pltpu.VMEM((HIST, D), jnp.float32),
        pltpu.VMEM((HIST, D), jnp.float32),
        pltpu.VMEM((_BPW, D), jnp.float32),
        pltpu.SemaphoreType.DMA,
        pltpu.SemaphoreType.DMA,
    ],
)
def _emb_kernel(items_hbm, table_hbm, out_hbm, idx_v, rows0, rows1, out_v,
                sem0, sem1):
    _emb_body(items_hbm, table_hbm, out_hbm, idx_v, rows0, rows1, out_v,
              sem0, sem1)


def kernel(items, table):
    items_flat = items.reshape(-1).astype(jnp.int32)
    return _emb_kernel(items_flat, table)


# speedup vs baseline: 14.6467x; 14.6467x over previous
import functools

import jax
import jax.numpy as jnp
from jax import lax
from jax.experimental import pallas as pl
from jax.experimental.pallas import tpu as pltpu
from jax.experimental.pallas import tpu_sc as plsc

BATCH = 4096
HIST = 200
D = 64
LANES = 16

_NC = 2
_NS = 16
_NW = _NC * _NS
_BPW = BATCH // _NW
_G0 = 128
_G1 = HIST - _G0
_UNROLL = 8


def _emb_body(items_hbm, table_hbm, out_hbm, idx_v, rows0, rows1, out_v,
              sem0, sem1):
    wid = lax.axis_index("s") * _NC + lax.axis_index("c")
    base = wid * (_BPW * HIST)

    pltpu.sync_copy(items_hbm.at[pl.ds(base, _BPW * HIST)], idx_v)

    def gather_descs(b, rows, sem):
        srcA = table_hbm.at[idx_v.at[pl.ds(b * HIST, _G0)]]
        srcB = table_hbm.at[idx_v.at[pl.ds(b * HIST + _G0, _G1)]]
        return ((srcA, rows.at[pl.ds(0, _G0)], sem),
                (srcB, rows.at[pl.ds(_G0, _G1)], sem))

    def start_gather(b, rows, sem):
        for desc in gather_descs(b, rows, sem):
            pltpu.async_copy(*desc)

    def wait_gather(b, rows, sem):
        for desc in gather_descs(b, rows, sem):
            pltpu.make_async_copy(*desc).wait()

    def phase(b, rows, sem):
        wait_gather(b, rows, sem)

        def acc_step(i, acc):
            a0, a1, a2, a3 = acc
            for j in range(_UNROLL):
                l = i * _UNROLL + j
                a0 = a0 + rows[l, pl.ds(0, LANES)]
                a1 = a1 + rows[l, pl.ds(LANES, LANES)]
                a2 = a2 + rows[l, pl.ds(2 * LANES, LANES)]
                a3 = a3 + rows[l, pl.ds(3 * LANES, LANES)]
            return (a0, a1, a2, a3)

        zero = jnp.zeros((LANES,), jnp.float32)
        acc = lax.fori_loop(0, HIST // _UNROLL, acc_step,
                            (zero, zero, zero, zero))
        scale = jnp.float32(1.0 / HIST)
        out_v[b, pl.ds(0, LANES)] = acc[0] * scale
        out_v[b, pl.ds(LANES, LANES)] = acc[1] * scale
        out_v[b, pl.ds(2 * LANES, LANES)] = acc[2] * scale
        out_v[b, pl.ds(3 * LANES, LANES)] = acc[3] * scale

        @pl.when(b + 2 < _BPW)
        def _():
            start_gather(b + 2, rows, sem)

    start_gather(0, rows0, sem0)
    start_gather(1, rows1, sem1)

    @pl.loop(0, _BPW // 2)
    def _(g):
        phase(2 * g, rows0, sem0)
        phase(2 * g + 1, rows1, sem1)

    pltpu.sync_copy(out_v, out_hbm.at[pl.ds(wid * _BPW, _BPW)])


@functools.partial(
    pl.kernel,
    out_type=jax.ShapeDtypeStruct((BATCH, D), jnp.float32),
    mesh=plsc.VectorSubcoreMesh(core_axis_name="c", subcore_axis_name="s"),
    compiler_params=pltpu.CompilerParams(use_tc_tiling_on_sc=False),
    scratch_types=[
        pltpu.VMEM((_BPW * HIST,), jnp.int32),
        pltpu.VMEM((HIST, D), jnp.float32),
        pltpu.VMEM((HIST, D), jnp.float32),
        pltpu.VMEM((_BPW, D), jnp.float32),
        pltpu.SemaphoreType.DMA,
        pltpu.SemaphoreType.DMA,
    ],
)
def _emb_kernel(items_hbm, table_hbm, out_hbm, idx_v, rows0, rows1, out_v,
                sem0, sem1):
    _emb_body(items_hbm, table_hbm, out_hbm, idx_v, rows0, rows1, out_v,
              sem0, sem1)


def kernel(items, table):
    items_flat = items.reshape(-1).astype(jnp.int32)
    return _emb_kernel(items_flat, table)


# speedup vs baseline: 18.1142x; 1.2367x over previous
import functools

import jax
import jax.numpy as jnp
from jax import lax
from jax.experimental import pallas as pl
from jax.experimental.pallas import tpu as pltpu
from jax.experimental.pallas import tpu_sc as plsc

BATCH = 4096
HIST = 200
D = 64
LANES = 16

_NC = 2
_NS = 16
_NW = _NC * _NS
_BPW = BATCH // _NW
_G0 = 128
_G1 = HIST - _G0
_UNROLL = 8
_NBUF = 4


def _emb_body(items_hbm, table_hbm, out_hbm, idx_v, rows0, rows1, rows2,
              rows3, out_v, sem0, sem1, sem2, sem3):
    wid = lax.axis_index("s") * _NC + lax.axis_index("c")
    base = wid * (_BPW * HIST)

    pltpu.sync_copy(items_hbm.at[pl.ds(base, _BPW * HIST)], idx_v)

    def gather_descs(b, rows, sem):
        srcA = table_hbm.at[idx_v.at[pl.ds(b * HIST, _G0)]]
        srcB = table_hbm.at[idx_v.at[pl.ds(b * HIST + _G0, _G1)]]
        return ((srcA, rows.at[pl.ds(0, _G0)], sem),
                (srcB, rows.at[pl.ds(_G0, _G1)], sem))

    def start_gather(b, rows, sem):
        for desc in gather_descs(b, rows, sem):
            pltpu.async_copy(*desc)

    def wait_gather(b, rows, sem):
        for desc in gather_descs(b, rows, sem):
            pltpu.make_async_copy(*desc).wait()

    def phase(b, rows, sem):
        wait_gather(b, rows, sem)

        def acc_step(i, acc):
            a0, a1, a2, a3 = acc
            for j in range(_UNROLL):
                l = i * _UNROLL + j
                a0 = a0 + rows[l, pl.ds(0, LANES)]
                a1 = a1 + rows[l, pl.ds(LANES, LANES)]
                a2 = a2 + rows[l, pl.ds(2 * LANES, LANES)]
                a3 = a3 + rows[l, pl.ds(3 * LANES, LANES)]
            return (a0, a1, a2, a3)

        zero = jnp.zeros((LANES,), jnp.float32)
        acc = lax.fori_loop(0, HIST // _UNROLL, acc_step,
                            (zero, zero, zero, zero))
        scale = jnp.float32(1.0 / HIST)
        out_v[b, pl.ds(0, LANES)] = acc[0] * scale
        out_v[b, pl.ds(LANES, LANES)] = acc[1] * scale
        out_v[b, pl.ds(2 * LANES, LANES)] = acc[2] * scale
        out_v[b, pl.ds(3 * LANES, LANES)] = acc[3] * scale

        @pl.when(b + _NBUF < _BPW)
        def _():
            start_gather(b + _NBUF, rows, sem)

    bufs = (rows0, rows1, rows2, rows3)
    sems = (sem0, sem1, sem2, sem3)
    for p in range(_NBUF):
        start_gather(p, bufs[p], sems[p])

    @pl.loop(0, _BPW // _NBUF)
    def _(g):
        for p in range(_NBUF):
            phase(_NBUF * g + p, bufs[p], sems[p])

    pltpu.sync_copy(out_v, out_hbm.at[pl.ds(wid * _BPW, _BPW)])


@functools.partial(
    pl.kernel,
    out_type=jax.ShapeDtypeStruct((BATCH, D), jnp.float32),
    mesh=plsc.VectorSubcoreMesh(core_axis_name="c", subcore_axis_name="s"),
    compiler_params=pltpu.CompilerParams(use_tc_tiling_on_sc=False),
    scratch_types=[
        pltpu.VMEM((_BPW * HIST,), jnp.int32),
        pltpu.VMEM((HIST, D), jnp.float32),
        pltpu.VMEM((HIST, D), jnp.float32),
        pltpu.VMEM((HIST, D), jnp.float32),
        pltpu.VMEM((HIST, D), jnp.float32),
        pltpu.VMEM((_BPW, D), jnp.float32),
        pltpu.SemaphoreType.DMA,
        pltpu.SemaphoreType.DMA,
        pltpu.SemaphoreType.DMA,
        pltpu.SemaphoreType.DMA,
    ],
)
def _emb_kernel(items_hbm, table_hbm, out_hbm, idx_v, rows0, rows1, rows2,
                rows3, out_v, sem0, sem1, sem2, sem3):
    _emb_body(items_hbm, table_hbm, out_hbm, idx_v, rows0, rows1, rows2,
              rows3, out_v, sem0, sem1, sem2, sem3)


def kernel(items, table):
    items_flat = items.reshape(-1).astype(jnp.int32)
    return _emb_kernel(items_flat, table)
